# Initial kernel scaffold; baseline (speedup 1.0000x reference)
import jax
import jax.numpy as jnp
from jax.experimental import pallas as pl


def kernel(user_ids, item_ids, user_emb_gmf, item_emb_gmf, user_emb_mlp, item_emb_mlp, W1, b1, W2, b2, W3, b3, Wp, bp):
    raise NotImplementedError("write your pallas kernel here")


# speedup vs baseline: 2.3451x; 2.3451x over previous
import functools

import jax
import jax.numpy as jnp
from jax import lax
from jax.experimental import pallas as pl
from jax.experimental.pallas import tpu as pltpu
from jax.experimental.pallas import tpu_sc as plsc

B = 16384
D = 128
H1, H2, H3 = 64, 32, 16
NC = 2
NS = 16
NW = NC * NS
BPW = B // NW
CHUNK = 128
NCHUNK = BPW // CHUNK


def _sc_gather_body(uids, iids, tug, tig, tum, tim,
                    oug, oig, oum, oim,
                    idx_u, idx_i, buf0, buf1,
                    gsem0, gsem1, wsem0, wsem1):
    wid = lax.axis_index("s") * NC + lax.axis_index("c")
    base = wid * BPW
    for c in range(NCHUNK):
        pltpu.sync_copy(uids.at[pl.ds(base + c * CHUNK, CHUNK)], idx_u.at[c])
        pltpu.sync_copy(iids.at[pl.ds(base + c * CHUNK, CHUNK)], idx_i.at[c])
    bufs = (buf0, buf1)
    gsems = (gsem0, gsem1)
    wsems = (wsem0, wsem1)
    tasks = []
    for tbl, out, idx in ((tug, oug, idx_u), (tig, oig, idx_i),
                          (tum, oum, idx_u), (tim, oim, idx_i)):
        for c in range(NCHUNK):
            tasks.append((tbl, out, idx, c))
    g_cp = [None, None]
    w_cp = [None, None]
    prev = None
    for k, (tbl, out, idx, c) in enumerate(tasks):
        slot = k % 2
        if w_cp[slot] is not None:
            w_cp[slot].wait()
        g_cp[slot] = pltpu.async_copy(tbl.at[idx.at[c]], bufs[slot], gsems[slot])
        if prev is not None:
            ps, pout, prow = prev
            g_cp[ps].wait()
            w_cp[ps] = pltpu.async_copy(bufs[ps], pout.at[pl.ds(prow, CHUNK)],
                                        wsems[ps])
        prev = (slot, out, base + c * CHUNK)
    ps, pout, prow = prev
    g_cp[ps].wait()
    w_cp[ps] = pltpu.async_copy(bufs[ps], pout.at[pl.ds(prow, CHUNK)], wsems[ps])
    for slot in (0, 1):
        if w_cp[slot] is not None:
            w_cp[slot].wait()


@functools.cache
def _sc_gather():
    return pl.kernel(
        _sc_gather_body,
        out_type=[jax.ShapeDtypeStruct((B, D), jnp.float32)] * 4,
        mesh=plsc.VectorSubcoreMesh(core_axis_name="c", subcore_axis_name="s",
                                    num_cores=NC, num_subcores=NS),
        scratch_types=[
            pltpu.VMEM((NCHUNK, CHUNK), jnp.int32),
            pltpu.VMEM((NCHUNK, CHUNK), jnp.int32),
            pltpu.VMEM((CHUNK, D), jnp.float32),
            pltpu.VMEM((CHUNK, D), jnp.float32),
            pltpu.SemaphoreType.DMA,
            pltpu.SemaphoreType.DMA,
            pltpu.SemaphoreType.DMA,
            pltpu.SemaphoreType.DMA,
        ],
    )


def _tc_body(ug, ig, um, im, w1u, w1i, b1, w2, b2, w3, b3, wpg, wph, bp, out):
    h = jnp.dot(um[...], w1u[...], preferred_element_type=jnp.float32)
    h = h + jnp.dot(im[...], w1i[...], preferred_element_type=jnp.float32)
    h = jnp.maximum(h + b1[...], 0.0)
    h = jnp.maximum(
        jnp.dot(h, w2[...], preferred_element_type=jnp.float32) + b2[...], 0.0)
    h = jnp.maximum(
        jnp.dot(h, w3[...], preferred_element_type=jnp.float32) + b3[...], 0.0)
    g = ug[...] * ig[...]
    s = (jnp.sum(g * wpg[...], axis=1, keepdims=True)
         + jnp.sum(h * wph[...], axis=1, keepdims=True) + bp[...])
    out[...] = jax.nn.sigmoid(s)[:, 0]


RBLK = 2048


def _tc_call(ug, ig, um, im, w1u, w1i, b1r, w2, b2r, w3, b3r, wpgr, wphr, bpr):
    rb = lambda i: (i, 0)
    z = lambda i: (0, 0)
    return pl.pallas_call(
        _tc_body,
        grid=(B // RBLK,),
        in_specs=[
            pl.BlockSpec((RBLK, D), rb),
            pl.BlockSpec((RBLK, D), rb),
            pl.BlockSpec((RBLK, D), rb),
            pl.BlockSpec((RBLK, D), rb),
            pl.BlockSpec((D, H1), z),
            pl.BlockSpec((D, H1), z),
            pl.BlockSpec((1, H1), z),
            pl.BlockSpec((H1, H2), z),
            pl.BlockSpec((1, H2), z),
            pl.BlockSpec((H2, H3), z),
            pl.BlockSpec((1, H3), z),
            pl.BlockSpec((1, D), z),
            pl.BlockSpec((1, H3), z),
            pl.BlockSpec((1, 1), z),
        ],
        out_specs=pl.BlockSpec((RBLK,), lambda i: (i,)),
        out_shape=jax.ShapeDtypeStruct((B,), jnp.float32),
    )(ug, ig, um, im, w1u, w1i, b1r, w2, b2r, w3, b3r, wpgr, wphr, bpr)


def kernel(user_ids, item_ids, user_emb_gmf, item_emb_gmf, user_emb_mlp,
           item_emb_mlp, W1, b1, W2, b2, W3, b3, Wp, bp):
    uids = user_ids.astype(jnp.int32)
    iids = item_ids.astype(jnp.int32)
    ug, ig, um, im = _sc_gather()(uids, iids, user_emb_gmf, item_emb_gmf,
                                  user_emb_mlp, item_emb_mlp)
    w1u = W1[:D]
    w1i = W1[D:]
    wpg = Wp[:D, 0].reshape(1, D)
    wph = Wp[D:, 0].reshape(1, H3)
    return _tc_call(ug, ig, um, im, w1u, w1i, b1.reshape(1, H1), W2,
                    b2.reshape(1, H2), W3, b3.reshape(1, H3), wpg, wph,
                    bp.reshape(1, 1))
